# baseline (speedup 1.0000x reference)
import functools

import jax
import jax.numpy as jnp
from jax.experimental import pallas as pl
from jax.experimental.pallas import tpu as pltpu

_ROWS = 4096
_CHUNK = 512


def _fused_body(starts_ref, ends_ref, inv_ref, x_ref, w1_ref, b1_ref,
                w2_ref, b2_ref, out_ref, acc_ref, w1bf_ref, *, rows, chunk):
    i = pl.program_id(0)
    nsteps = pl.num_programs(0)

    @pl.when(i == 0)
    def _init():
        w1bf_ref[...] = w1_ref[...].astype(jnp.bfloat16)
        acc_ref[...] = jnp.zeros_like(acc_ref)

    part = jnp.zeros_like(acc_ref)
    for c in range(rows // chunk):
        xc = x_ref[c * chunk:(c + 1) * chunk, :].astype(jnp.bfloat16)
        h = jnp.dot(xc, w1bf_ref[...], preferred_element_type=jnp.float32)
        h = jnp.maximum(h + b1_ref[0:1, :], 0.0)
        base = i * rows + c * chunk
        gidx = base + jax.lax.broadcasted_iota(jnp.int32, (1, chunk), 1)
        mask = (gidx >= starts_ref[:, 0:1]) & (gidx < ends_ref[:, 0:1])
        onehot = jnp.where(mask, 1.0, 0.0).astype(jnp.bfloat16)
        part += jnp.dot(onehot, h.astype(jnp.bfloat16),
                        preferred_element_type=jnp.float32)

    acc_ref[...] += part

    @pl.when(i == nsteps - 1)
    def _epilogue():
        means = acc_ref[...] * inv_ref[:, 0:1]
        head = jnp.dot(means, w2_ref[...],
                       preferred_element_type=jnp.float32)
        out_ref[...] = jnp.tanh(head + b2_ref[0:1, :])


def kernel(x, n_instances, W1, b1, W2, b2):
    n, d_in = x.shape
    d_h = W1.shape[1]
    d_out = W2.shape[1]
    b = n_instances.shape[0]
    rows = _ROWS
    nsteps = n // rows

    counts = n_instances.astype(jnp.int32)
    ends = jnp.cumsum(counts)
    starts = ends - counts
    inv = 1.0 / jnp.maximum(counts, 1).astype(jnp.float32)
    starts2d = jnp.broadcast_to(starts[:, None], (b, 128))
    ends2d = jnp.broadcast_to(ends[:, None], (b, 128))
    inv2d = jnp.broadcast_to(inv[:, None], (b, 128))
    b1_2d = jnp.broadcast_to(b1[None, :], (8, d_h))
    b2_2d = jnp.broadcast_to(b2[None, :], (8, d_out))

    in_specs = [
        pl.BlockSpec((b, 128), lambda i: (0, 0)),
        pl.BlockSpec((b, 128), lambda i: (0, 0)),
        pl.BlockSpec((b, 128), lambda i: (0, 0)),
        pl.BlockSpec((rows, d_in), lambda i: (i, 0)),
        pl.BlockSpec((d_in, d_h), lambda i: (0, 0)),
        pl.BlockSpec((8, d_h), lambda i: (0, 0)),
        pl.BlockSpec((d_h, d_out), lambda i: (0, 0)),
        pl.BlockSpec((8, d_out), lambda i: (0, 0)),
    ]

    return pl.pallas_call(
        functools.partial(_fused_body, rows=rows, chunk=_CHUNK),
        grid=(nsteps,),
        in_specs=in_specs,
        out_specs=pl.BlockSpec((b, d_out), lambda i: (0, 0)),
        out_shape=jax.ShapeDtypeStruct((b, d_out), jnp.float32),
        scratch_shapes=[pltpu.VMEM((b, d_h), jnp.float32),
                        pltpu.VMEM((d_in, d_h), jnp.bfloat16)],
        compiler_params=pltpu.CompilerParams(
            dimension_semantics=("arbitrary",),
        ),
    )(starts2d, ends2d, inv2d, x, W1, b1_2d, W2, b2_2d)


# speedup vs baseline: 1.0887x; 1.0887x over previous
import functools

import jax
import jax.numpy as jnp
from jax.experimental import pallas as pl
from jax.experimental.pallas import tpu as pltpu

_ROWS = 4096
_CHUNK = 512


def _fused_body(starts_ref, ends_ref, inv_ref, x_ref, w1_ref, b1_ref,
                w2_ref, b2_ref, out_ref, acc_ref, w1bf_ref, *, rows, chunk):
    i = pl.program_id(0)
    nsteps = pl.num_programs(0)

    @pl.when(i == 0)
    def _init():
        w1bf_ref[...] = w1_ref[...].astype(jnp.bfloat16)
        acc_ref[...] = jnp.zeros_like(acc_ref)

    xc = x_ref[...].astype(jnp.bfloat16)
    h = jnp.dot(xc, w1bf_ref[...],
                preferred_element_type=jnp.float32).astype(jnp.bfloat16)
    h = jnp.maximum(h + b1_ref[0:1, :].astype(jnp.bfloat16), 0)
    base = i * rows
    gidx = base + jax.lax.broadcasted_iota(jnp.int32, (1, rows), 1)
    mask = (gidx >= starts_ref[:, 0:1]) & (gidx < ends_ref[:, 0:1])
    onehot = jnp.where(mask, 1.0, 0.0).astype(jnp.bfloat16)
    acc_ref[...] += jnp.dot(onehot, h,
                            preferred_element_type=jnp.float32)

    @pl.when(i == nsteps - 1)
    def _epilogue():
        means = acc_ref[...] * inv_ref[:, 0:1]
        head = jnp.dot(means, w2_ref[...],
                       preferred_element_type=jnp.float32)
        out_ref[...] = jnp.tanh(head + b2_ref[0:1, :])


def kernel(x, n_instances, W1, b1, W2, b2):
    n, d_in = x.shape
    d_h = W1.shape[1]
    d_out = W2.shape[1]
    b = n_instances.shape[0]
    rows = _ROWS
    nsteps = n // rows

    counts = n_instances.astype(jnp.int32)
    ends = jnp.cumsum(counts)
    starts = ends - counts
    inv = 1.0 / jnp.maximum(counts, 1).astype(jnp.float32)
    starts2d = jnp.broadcast_to(starts[:, None], (b, 128))
    ends2d = jnp.broadcast_to(ends[:, None], (b, 128))
    inv2d = jnp.broadcast_to(inv[:, None], (b, 128))
    b1_2d = jnp.broadcast_to(b1[None, :], (8, d_h))
    b2_2d = jnp.broadcast_to(b2[None, :], (8, d_out))

    in_specs = [
        pl.BlockSpec((b, 128), lambda i: (0, 0)),
        pl.BlockSpec((b, 128), lambda i: (0, 0)),
        pl.BlockSpec((b, 128), lambda i: (0, 0)),
        pl.BlockSpec((rows, d_in), lambda i: (i, 0)),
        pl.BlockSpec((d_in, d_h), lambda i: (0, 0)),
        pl.BlockSpec((8, d_h), lambda i: (0, 0)),
        pl.BlockSpec((d_h, d_out), lambda i: (0, 0)),
        pl.BlockSpec((8, d_out), lambda i: (0, 0)),
    ]

    return pl.pallas_call(
        functools.partial(_fused_body, rows=rows, chunk=_CHUNK),
        grid=(nsteps,),
        in_specs=in_specs,
        out_specs=pl.BlockSpec((b, d_out), lambda i: (0, 0)),
        out_shape=jax.ShapeDtypeStruct((b, d_out), jnp.float32),
        scratch_shapes=[pltpu.VMEM((b, d_h), jnp.float32),
                        pltpu.VMEM((d_in, d_h), jnp.bfloat16)],
        compiler_params=pltpu.CompilerParams(
            dimension_semantics=("arbitrary",),
        ),
    )(starts2d, ends2d, inv2d, x, W1, b1_2d, W2, b2_2d)


# speedup vs baseline: 1.1154x; 1.0245x over previous
import functools

import jax
import jax.numpy as jnp
from jax.experimental import pallas as pl
from jax.experimental.pallas import tpu as pltpu

_ROWS = 2048
_NBUF = 4


def _fused_body(starts_ref, ends_ref, inv_ref, x_hbm, w1_ref, b1_ref,
                w2_ref, b2_ref, out_ref, acc_ref, w1bf_ref, xbuf, sems,
                *, rows, nbuf):
    i = pl.program_id(0)
    nsteps = pl.num_programs(0)

    def copy_in(w, slot):
        pltpu.make_async_copy(
            x_hbm.at[pl.ds(w * rows, rows), :],
            xbuf.at[slot],
            sems.at[slot],
        ).start()

    @pl.when(i == 0)
    def _init():
        w1bf_ref[...] = w1_ref[...].astype(jnp.bfloat16)
        acc_ref[...] = jnp.zeros_like(acc_ref)
        for k in range(nbuf - 1):
            copy_in(k, k)

    nxt = i + nbuf - 1

    @pl.when(nxt < nsteps)
    def _prefetch():
        copy_in(nxt, nxt % nbuf)

    slot = i % nbuf
    pltpu.make_async_copy(
        x_hbm.at[pl.ds(i * rows, rows), :],
        xbuf.at[slot],
        sems.at[slot],
    ).wait()

    xc = xbuf[slot].astype(jnp.bfloat16)
    h = jnp.dot(xc, w1bf_ref[...],
                preferred_element_type=jnp.float32).astype(jnp.bfloat16)
    h = jnp.maximum(h + b1_ref[0:1, :].astype(jnp.bfloat16), 0)
    gidx = i * rows + jax.lax.broadcasted_iota(jnp.int32, (1, rows), 1)
    mask = (gidx >= starts_ref[:, 0:1]) & (gidx < ends_ref[:, 0:1])
    onehot = jnp.where(mask, 1.0, 0.0).astype(jnp.bfloat16)
    acc_ref[...] += jnp.dot(onehot, h, preferred_element_type=jnp.float32)

    @pl.when(i == nsteps - 1)
    def _epilogue():
        means = acc_ref[...] * inv_ref[:, 0:1]
        head = jnp.dot(means, w2_ref[...],
                       preferred_element_type=jnp.float32)
        out_ref[...] = jnp.tanh(head + b2_ref[0:1, :])


def kernel(x, n_instances, W1, b1, W2, b2):
    n, d_in = x.shape
    d_h = W1.shape[1]
    d_out = W2.shape[1]
    b = n_instances.shape[0]
    rows = _ROWS
    nbuf = _NBUF
    nsteps = n // rows

    counts = n_instances.astype(jnp.int32)
    ends = jnp.cumsum(counts)
    starts = ends - counts
    inv = 1.0 / jnp.maximum(counts, 1).astype(jnp.float32)
    starts2d = jnp.broadcast_to(starts[:, None], (b, 128))
    ends2d = jnp.broadcast_to(ends[:, None], (b, 128))
    inv2d = jnp.broadcast_to(inv[:, None], (b, 128))
    b1_2d = jnp.broadcast_to(b1[None, :], (8, d_h))
    b2_2d = jnp.broadcast_to(b2[None, :], (8, d_out))

    in_specs = [
        pl.BlockSpec((b, 128), lambda i: (0, 0)),
        pl.BlockSpec((b, 128), lambda i: (0, 0)),
        pl.BlockSpec((b, 128), lambda i: (0, 0)),
        pl.BlockSpec(memory_space=pl.ANY),
        pl.BlockSpec((d_in, d_h), lambda i: (0, 0)),
        pl.BlockSpec((8, d_h), lambda i: (0, 0)),
        pl.BlockSpec((d_h, d_out), lambda i: (0, 0)),
        pl.BlockSpec((8, d_out), lambda i: (0, 0)),
    ]

    return pl.pallas_call(
        functools.partial(_fused_body, rows=rows, nbuf=nbuf),
        grid=(nsteps,),
        in_specs=in_specs,
        out_specs=pl.BlockSpec((b, d_out), lambda i: (0, 0)),
        out_shape=jax.ShapeDtypeStruct((b, d_out), jnp.float32),
        scratch_shapes=[
            pltpu.VMEM((b, d_h), jnp.float32),
            pltpu.VMEM((d_in, d_h), jnp.bfloat16),
            pltpu.VMEM((nbuf, rows, d_in), jnp.float32),
            pltpu.SemaphoreType.DMA((nbuf,)),
        ],
        compiler_params=pltpu.CompilerParams(
            dimension_semantics=("arbitrary",),
        ),
    )(starts2d, ends2d, inv2d, x, W1, b1_2d, W2, b2_2d)
